# Initial kernel scaffold; baseline (speedup 1.0000x reference)
import jax
import jax.numpy as jnp
from jax.experimental import pallas as pl


def kernel(x, user_table, item_table):
    raise NotImplementedError("write your pallas kernel here")


# speedup vs baseline: 1.2964x; 1.2964x over previous
import jax
import jax.numpy as jnp
from jax import lax
from jax.experimental import pallas as pl
from jax.experimental.pallas import tpu as pltpu, tpu_sc as plsc

BATCH = 16384
EMBED_K = 128
CHUNK = 128

_info = plsc.get_sparse_core_info()
NC, NS = _info.num_cores, _info.num_subcores
NW = NC * NS
B_PER_W = BATCH // NW
CHUNKS_PER_W = B_PER_W // CHUNK

_mesh = plsc.VectorSubcoreMesh(core_axis_name="c", subcore_axis_name="s")


@jax.jit
def _gather2(user_idx, item_idx, user_table, item_table):
    @pl.kernel(
        mesh=_mesh,
        out_type=(
            jax.ShapeDtypeStruct((BATCH, EMBED_K), jnp.float32),
            jax.ShapeDtypeStruct((BATCH, EMBED_K), jnp.float32),
        ),
        scratch_types=[
            pltpu.VMEM((CHUNKS_PER_W, CHUNK), jnp.int32),
            pltpu.VMEM((CHUNKS_PER_W, CHUNK), jnp.int32),
            pltpu.VMEM((CHUNK, EMBED_K), jnp.float32),
            pltpu.SemaphoreType.DMA,
        ],
    )
    def k(uidx_hbm, iidx_hbm, utab_hbm, itab_hbm, uout_hbm, iout_hbm,
          idx_u, idx_i, rows, sem):
        wid = lax.axis_index("s") * NC + lax.axis_index("c")
        pltpu.sync_copy(uidx_hbm.at[pl.ds(wid * CHUNKS_PER_W, CHUNKS_PER_W)], idx_u)
        pltpu.sync_copy(iidx_hbm.at[pl.ds(wid * CHUNKS_PER_W, CHUNKS_PER_W)], idx_i)
        base = wid * B_PER_W
        for j in range(CHUNKS_PER_W):
            pltpu.async_copy(utab_hbm.at[idx_u.at[j]], rows, sem).wait()
            pltpu.sync_copy(rows, uout_hbm.at[pl.ds(base + j * CHUNK, CHUNK)])
        for j in range(CHUNKS_PER_W):
            pltpu.async_copy(itab_hbm.at[idx_i.at[j]], rows, sem).wait()
            pltpu.sync_copy(rows, iout_hbm.at[pl.ds(base + j * CHUNK, CHUNK)])

    return k(user_idx, item_idx, user_table, item_table)


def kernel(x, user_table, item_table):
    user_idx = x[:, 0].reshape(BATCH // CHUNK, CHUNK)
    item_idx = x[:, 1].reshape(BATCH // CHUNK, CHUNK)
    return _gather2(user_idx, item_idx, user_table, item_table)


# speedup vs baseline: 1.5198x; 1.1722x over previous
import jax
import jax.numpy as jnp
from jax import lax
from jax.experimental import pallas as pl
from jax.experimental.pallas import tpu as pltpu, tpu_sc as plsc

BATCH = 16384
EMBED_K = 128
CHUNK = 128
NBUF = 4

_info = plsc.get_sparse_core_info()
NC, NS = _info.num_cores, _info.num_subcores
NW = NC * NS
B_PER_W = BATCH // NW
CHUNKS_PER_W = B_PER_W // CHUNK

_mesh = plsc.VectorSubcoreMesh(core_axis_name="c", subcore_axis_name="s")


@jax.jit
def _gather2(user_idx, item_idx, user_table, item_table):
    @pl.kernel(
        mesh=_mesh,
        out_type=(
            jax.ShapeDtypeStruct((BATCH, EMBED_K), jnp.float32),
            jax.ShapeDtypeStruct((BATCH, EMBED_K), jnp.float32),
        ),
        scratch_types=[
            pltpu.VMEM((CHUNKS_PER_W, CHUNK), jnp.int32),
            pltpu.VMEM((CHUNKS_PER_W, CHUNK), jnp.int32),
            pltpu.VMEM((NBUF, CHUNK, EMBED_K), jnp.float32),
            pltpu.SemaphoreType.DMA((NBUF,)),
            pltpu.SemaphoreType.DMA((NBUF,)),
        ],
    )
    def k(uidx_hbm, iidx_hbm, utab_hbm, itab_hbm, uout_hbm, iout_hbm,
          idx_u, idx_i, rows, gsem, osem):
        wid = lax.axis_index("s") * NC + lax.axis_index("c")
        pltpu.sync_copy(uidx_hbm.at[pl.ds(wid * CHUNKS_PER_W, CHUNKS_PER_W)], idx_u)
        pltpu.sync_copy(iidx_hbm.at[pl.ds(wid * CHUNKS_PER_W, CHUNKS_PER_W)], idx_i)
        base = wid * B_PER_W

        def fire_gather(c, buf):
            if c < CHUNKS_PER_W:
                src = utab_hbm.at[idx_u.at[c]]
            else:
                src = itab_hbm.at[idx_i.at[c - CHUNKS_PER_W]]
            return pltpu.async_copy(src, rows.at[buf], gsem.at[buf])

        def fire_out(c, buf):
            if c < CHUNKS_PER_W:
                dst = uout_hbm.at[pl.ds(base + c * CHUNK, CHUNK)]
            else:
                dst = iout_hbm.at[pl.ds(base + (c - CHUNKS_PER_W) * CHUNK, CHUNK)]
            return pltpu.async_copy(rows.at[buf], dst, osem.at[buf])

        nchunks = 2 * CHUNKS_PER_W
        gathers = [fire_gather(c, c % NBUF) for c in range(NBUF)]
        outs = [None] * NBUF
        for c in range(nchunks):
            buf = c % NBUF
            gathers[buf].wait()
            outs[buf] = fire_out(c, buf)
            if c + NBUF < nchunks:
                outs[buf].wait()
                gathers[buf] = fire_gather(c + NBUF, buf)
        for c in range(nchunks - NBUF, nchunks):
            outs[c % NBUF].wait()

    return k(user_idx, item_idx, user_table, item_table)


def kernel(x, user_table, item_table):
    user_idx = x[:, 0].reshape(BATCH // CHUNK, CHUNK)
    item_idx = x[:, 1].reshape(BATCH // CHUNK, CHUNK)
    return _gather2(user_idx, item_idx, user_table, item_table)


# speedup vs baseline: 1.5697x; 1.0328x over previous
import jax
import jax.numpy as jnp
from jax import lax
from jax.experimental import pallas as pl
from jax.experimental.pallas import tpu as pltpu, tpu_sc as plsc

BATCH = 16384
EMBED_K = 128
CHUNK = 128
NBUF = 7

_info = plsc.get_sparse_core_info()
NC, NS = _info.num_cores, _info.num_subcores
NW = NC * NS
B_PER_W = BATCH // NW
CHUNKS_PER_W = B_PER_W // CHUNK

_mesh = plsc.VectorSubcoreMesh(core_axis_name="c", subcore_axis_name="s")


@jax.jit
def _gather2(user_idx, item_idx, user_table, item_table):
    @pl.kernel(
        mesh=_mesh,
        out_type=(
            jax.ShapeDtypeStruct((BATCH, EMBED_K), jnp.float32),
            jax.ShapeDtypeStruct((BATCH, EMBED_K), jnp.float32),
        ),
        scratch_types=[
            pltpu.VMEM((CHUNKS_PER_W, CHUNK), jnp.int32),
            pltpu.VMEM((CHUNKS_PER_W, CHUNK), jnp.int32),
            pltpu.VMEM((NBUF, CHUNK, EMBED_K), jnp.float32),
            pltpu.SemaphoreType.DMA((NBUF,)),
            pltpu.SemaphoreType.DMA((NBUF,)),
            pltpu.SemaphoreType.DMA,
        ],
    )
    def k(uidx_hbm, iidx_hbm, utab_hbm, itab_hbm, uout_hbm, iout_hbm,
          idx_u, idx_i, rows, gsem, osem, isem):
        wid = lax.axis_index("s") * NC + lax.axis_index("c")
        iu = pltpu.async_copy(
            uidx_hbm.at[pl.ds(wid * CHUNKS_PER_W, CHUNKS_PER_W)], idx_u, isem)
        ii = pltpu.async_copy(
            iidx_hbm.at[pl.ds(wid * CHUNKS_PER_W, CHUNKS_PER_W)], idx_i, isem)
        iu.wait()
        ii.wait()
        base = wid * B_PER_W

        def fire_gather(c, buf):
            if c < CHUNKS_PER_W:
                src = utab_hbm.at[idx_u.at[c]]
            else:
                src = itab_hbm.at[idx_i.at[c - CHUNKS_PER_W]]
            return pltpu.async_copy(src, rows.at[buf], gsem.at[buf])

        def fire_out(c, buf):
            if c < CHUNKS_PER_W:
                dst = uout_hbm.at[pl.ds(base + c * CHUNK, CHUNK)]
            else:
                dst = iout_hbm.at[pl.ds(base + (c - CHUNKS_PER_W) * CHUNK, CHUNK)]
            return pltpu.async_copy(rows.at[buf], dst, osem.at[buf])

        nchunks = 2 * CHUNKS_PER_W
        gathers = [fire_gather(c, c % NBUF) for c in range(NBUF)]
        outs = [None] * NBUF
        for c in range(nchunks):
            buf = c % NBUF
            gathers[buf].wait()
            outs[buf] = fire_out(c, buf)
            if c + NBUF < nchunks:
                outs[buf].wait()
                gathers[buf] = fire_gather(c + NBUF, buf)
        for c in range(nchunks - NBUF, nchunks):
            outs[c % NBUF].wait()

    return k(user_idx, item_idx, user_table, item_table)


def kernel(x, user_table, item_table):
    user_idx = x[:, 0].reshape(BATCH // CHUNK, CHUNK)
    item_idx = x[:, 1].reshape(BATCH // CHUNK, CHUNK)
    return _gather2(user_idx, item_idx, user_table, item_table)
